# Initial kernel scaffold; baseline (speedup 1.0000x reference)
import jax
import jax.numpy as jnp
from jax.experimental import pallas as pl


def kernel(rects, conf):
    raise NotImplementedError("write your pallas kernel here")


# speedup vs baseline: 1.6376x; 1.6376x over previous
import jax
import jax.numpy as jnp
import numpy as np
from jax.experimental import pallas as pl

_W0 = 16.0
_H0 = 16.0
_ALPHA = 1.5
_LOG_ALPHA = float(np.log(1.5))
_GAMMA = 0.5
_BX = 0.5
_BY = 0.5
_B = 1 << 14
_M = 1 << 15

_TI = 256
_TJ = 1024


def _cell_keys(x1, y1, x2, y2):
    w = jnp.maximum(x2 - x1, 1e-6)
    h = jnp.maximum(y2 - y1, 1e-6)
    cx = (x1 + x2) * 0.5
    cy = (y1 + y2) * 0.5
    iw = jnp.round(jnp.log(w / _W0) / _LOG_ALPHA)
    ih = jnp.round(jnp.log(h / _H0) / _LOG_ALPHA)
    cell_w = _GAMMA * _W0 * jnp.power(_ALPHA, iw)
    cell_h = _GAMMA * _H0 * jnp.power(_ALPHA, ih)
    ix = jnp.round((cx - _BX * cell_w) / cell_w)
    iy = jnp.round((cy - _BY * cell_h) / cell_h)

    def enc(v):
        return jnp.clip(v.astype(jnp.int32) + _B, 0, _M - 1)

    khi = enc(iw) * _M + enc(ih)
    klo = enc(ix) * _M + enc(iy)
    return khi, klo


def _suppress_kernel(rects_ref, rects_t_ref, conf_ref, conf_t_ref, sup_ref):
    c = pl.program_id(1)

    @pl.when(c == 0)
    def _init():
        sup_ref[...] = jnp.zeros(sup_ref.shape, sup_ref.dtype)

    r = rects_ref[...]
    khi_i, klo_i = _cell_keys(r[:, 0:1], r[:, 1:2], r[:, 2:3], r[:, 3:4])
    rt = rects_t_ref[...]
    khi_j, klo_j = _cell_keys(rt[0:1, :], rt[1:2, :], rt[2:3, :], rt[3:4, :])
    ci = conf_ref[...]
    cj = conf_t_ref[...]
    dom = (khi_i == khi_j) & (klo_i == klo_j) & (cj > ci)
    any_dom = jnp.max(dom.astype(jnp.float32), axis=1, keepdims=True)
    sup_ref[...] = jnp.maximum(sup_ref[...], any_dom)


def kernel(rects, conf):
    n = rects.shape[0]
    rects = rects.astype(jnp.float32)
    conf = conf.astype(jnp.float32)

    nr = ((n + _TI - 1) // _TI) * _TI
    nc = ((n + _TJ - 1) // _TJ) * _TJ

    rects_r = jnp.pad(rects, ((0, nr - n), (0, 0)))
    conf_r = jnp.pad(conf, (0, nr - n)).reshape(nr, 1)
    rects_c = jnp.pad(rects, ((0, nc - n), (0, 0)))
    conf_c = jnp.pad(conf, (0, nc - n), constant_values=-1.0).reshape(1, nc)
    rects_t = rects_c.T

    grid = (nr // _TI, nc // _TJ)
    sup = pl.pallas_call(
        _suppress_kernel,
        grid=grid,
        in_specs=[
            pl.BlockSpec((_TI, 4), lambda r, c: (r, r * 0)),
            pl.BlockSpec((4, _TJ), lambda r, c: (c * 0, c)),
            pl.BlockSpec((_TI, 1), lambda r, c: (r, r * 0)),
            pl.BlockSpec((1, _TJ), lambda r, c: (c * 0, c)),
        ],
        out_specs=pl.BlockSpec((_TI, 1), lambda r, c: (r, r * 0)),
        out_shape=jax.ShapeDtypeStruct((nr, 1), jnp.float32),
    )(rects_r, rects_t, conf_r, conf_c)

    keep = 1.0 - sup[:n, 0]
    out = jnp.concatenate([rects * keep[:, None], (conf * keep)[:, None]], axis=1)
    return out


# speedup vs baseline: 2.7670x; 1.6897x over previous
import jax
import jax.numpy as jnp
import numpy as np
from jax.experimental import pallas as pl

_W0 = 16.0
_H0 = 16.0
_ALPHA = 1.5
_LOG_ALPHA = float(np.log(1.5))
_GAMMA = 0.5
_BX = 0.5
_BY = 0.5
_B = 1 << 14
_M = 1 << 15

_TI = 512
_TJ = 2048


def _cell_keys(x1, y1, x2, y2):
    w = jnp.maximum(x2 - x1, 1e-6)
    h = jnp.maximum(y2 - y1, 1e-6)
    cx = (x1 + x2) * 0.5
    cy = (y1 + y2) * 0.5
    iw = jnp.round(jnp.log(w / _W0) / _LOG_ALPHA)
    ih = jnp.round(jnp.log(h / _H0) / _LOG_ALPHA)
    cell_w = _GAMMA * _W0 * jnp.power(_ALPHA, iw)
    cell_h = _GAMMA * _H0 * jnp.power(_ALPHA, ih)
    ix = jnp.round((cx - _BX * cell_w) / cell_w)
    iy = jnp.round((cy - _BY * cell_h) / cell_h)

    def enc(v):
        return jnp.clip(v.astype(jnp.int32) + _B, 0, _M - 1)

    khi = enc(iw) * _M + enc(ih)
    klo = enc(ix) * _M + enc(iy)
    return khi, klo


def _suppress_kernel(rects_ref, rects_t_ref, conf_ref, conf_t_ref, sup_ref):
    c = pl.program_id(1)

    @pl.when(c == 0)
    def _init():
        sup_ref[...] = jnp.zeros(sup_ref.shape, sup_ref.dtype)

    r = rects_ref[...]
    khi_i, klo_i = _cell_keys(r[:, 0:1], r[:, 1:2], r[:, 2:3], r[:, 3:4])
    rt = rects_t_ref[...]
    khi_j, klo_j = _cell_keys(rt[0:1, :], rt[1:2, :], rt[2:3, :], rt[3:4, :])
    ci = conf_ref[...]
    cj = conf_t_ref[...]
    dom = (khi_i == khi_j) & (klo_i == klo_j) & (cj > ci)
    any_dom = jnp.max(dom.astype(jnp.float32), axis=1, keepdims=True)
    sup_ref[...] = jnp.maximum(sup_ref[...], any_dom)


def kernel(rects, conf):
    n = rects.shape[0]
    rects = rects.astype(jnp.float32)
    conf = conf.astype(jnp.float32)

    nr = ((n + _TI - 1) // _TI) * _TI
    nc = ((n + _TJ - 1) // _TJ) * _TJ

    rects_r = jnp.pad(rects, ((0, nr - n), (0, 0)))
    conf_r = jnp.pad(conf, (0, nr - n)).reshape(nr, 1)
    rects_c = jnp.pad(rects, ((0, nc - n), (0, 0)))
    conf_c = jnp.pad(conf, (0, nc - n), constant_values=-1.0).reshape(1, nc)
    rects_t = rects_c.T

    grid = (nr // _TI, nc // _TJ)
    sup = pl.pallas_call(
        _suppress_kernel,
        grid=grid,
        in_specs=[
            pl.BlockSpec((_TI, 4), lambda r, c: (r, r * 0)),
            pl.BlockSpec((4, _TJ), lambda r, c: (c * 0, c)),
            pl.BlockSpec((_TI, 1), lambda r, c: (r, r * 0)),
            pl.BlockSpec((1, _TJ), lambda r, c: (c * 0, c)),
        ],
        out_specs=pl.BlockSpec((_TI, 1), lambda r, c: (r, r * 0)),
        out_shape=jax.ShapeDtypeStruct((nr, 1), jnp.float32),
    )(rects_r, rects_t, conf_r, conf_c)

    keep = 1.0 - sup[:n, 0]
    out = jnp.concatenate([rects * keep[:, None], (conf * keep)[:, None]], axis=1)
    return out


# speedup vs baseline: 3.4653x; 1.2524x over previous
import jax
import jax.numpy as jnp
import numpy as np
from jax.experimental import pallas as pl

_W0 = 16.0
_H0 = 16.0
_ALPHA = 1.5
_LOG_ALPHA = float(np.log(1.5))
_GAMMA = 0.5
_BX = 0.5
_BY = 0.5
_B = 1 << 14
_M = 1 << 15

_TI = 1024
_TJ = 4096


def _cell_keys(x1, y1, x2, y2):
    w = jnp.maximum(x2 - x1, 1e-6)
    h = jnp.maximum(y2 - y1, 1e-6)
    cx = (x1 + x2) * 0.5
    cy = (y1 + y2) * 0.5
    iw = jnp.round(jnp.log(w / _W0) / _LOG_ALPHA)
    ih = jnp.round(jnp.log(h / _H0) / _LOG_ALPHA)
    cell_w = _GAMMA * _W0 * jnp.power(_ALPHA, iw)
    cell_h = _GAMMA * _H0 * jnp.power(_ALPHA, ih)
    ix = jnp.round((cx - _BX * cell_w) / cell_w)
    iy = jnp.round((cy - _BY * cell_h) / cell_h)

    def enc(v):
        return jnp.clip(v.astype(jnp.int32) + _B, 0, _M - 1)

    khi = enc(iw) * _M + enc(ih)
    klo = enc(ix) * _M + enc(iy)
    return khi, klo


def _suppress_kernel(rects_ref, rects_t_ref, conf_ref, conf_t_ref, sup_ref):
    c = pl.program_id(1)

    @pl.when(c == 0)
    def _init():
        sup_ref[...] = jnp.zeros(sup_ref.shape, sup_ref.dtype)

    r = rects_ref[...]
    khi_i, klo_i = _cell_keys(r[:, 0:1], r[:, 1:2], r[:, 2:3], r[:, 3:4])
    rt = rects_t_ref[...]
    khi_j, klo_j = _cell_keys(rt[0:1, :], rt[1:2, :], rt[2:3, :], rt[3:4, :])
    ci = conf_ref[...]
    cj = conf_t_ref[...]
    dom = (khi_i == khi_j) & (klo_i == klo_j) & (cj > ci)
    any_dom = jnp.max(dom.astype(jnp.float32), axis=1, keepdims=True)
    sup_ref[...] = jnp.maximum(sup_ref[...], any_dom)


def kernel(rects, conf):
    n = rects.shape[0]
    rects = rects.astype(jnp.float32)
    conf = conf.astype(jnp.float32)

    nr = ((n + _TI - 1) // _TI) * _TI
    nc = ((n + _TJ - 1) // _TJ) * _TJ

    rects_r = jnp.pad(rects, ((0, nr - n), (0, 0)))
    conf_r = jnp.pad(conf, (0, nr - n)).reshape(nr, 1)
    rects_c = jnp.pad(rects, ((0, nc - n), (0, 0)))
    conf_c = jnp.pad(conf, (0, nc - n), constant_values=-1.0).reshape(1, nc)
    rects_t = rects_c.T

    grid = (nr // _TI, nc // _TJ)
    sup = pl.pallas_call(
        _suppress_kernel,
        grid=grid,
        in_specs=[
            pl.BlockSpec((_TI, 4), lambda r, c: (r, r * 0)),
            pl.BlockSpec((4, _TJ), lambda r, c: (c * 0, c)),
            pl.BlockSpec((_TI, 1), lambda r, c: (r, r * 0)),
            pl.BlockSpec((1, _TJ), lambda r, c: (c * 0, c)),
        ],
        out_specs=pl.BlockSpec((_TI, 1), lambda r, c: (r, r * 0)),
        out_shape=jax.ShapeDtypeStruct((nr, 1), jnp.float32),
    )(rects_r, rects_t, conf_r, conf_c)

    keep = 1.0 - sup[:n, 0]
    out = jnp.concatenate([rects * keep[:, None], (conf * keep)[:, None]], axis=1)
    return out


# speedup vs baseline: 3.4953x; 1.0086x over previous
import jax
import jax.numpy as jnp
import numpy as np
from jax.experimental import pallas as pl

_W0 = 16.0
_H0 = 16.0
_ALPHA = 1.5
_LOG_ALPHA = float(np.log(1.5))
_GAMMA = 0.5
_BX = 0.5
_BY = 0.5
_B = 1 << 14
_M = 1 << 15

_TI = 2048
_TJ = 4096


def _cell_keys(x1, y1, x2, y2):
    w = jnp.maximum(x2 - x1, 1e-6)
    h = jnp.maximum(y2 - y1, 1e-6)
    cx = (x1 + x2) * 0.5
    cy = (y1 + y2) * 0.5
    iw = jnp.round(jnp.log(w / _W0) / _LOG_ALPHA)
    ih = jnp.round(jnp.log(h / _H0) / _LOG_ALPHA)
    cell_w = _GAMMA * _W0 * jnp.power(_ALPHA, iw)
    cell_h = _GAMMA * _H0 * jnp.power(_ALPHA, ih)
    ix = jnp.round((cx - _BX * cell_w) / cell_w)
    iy = jnp.round((cy - _BY * cell_h) / cell_h)

    def enc(v):
        return jnp.clip(v.astype(jnp.int32) + _B, 0, _M - 1)

    khi = enc(iw) * _M + enc(ih)
    klo = enc(ix) * _M + enc(iy)
    return khi, klo


def _suppress_kernel(rects_ref, rects_t_ref, conf_ref, conf_t_ref, sup_ref):
    c = pl.program_id(1)

    @pl.when(c == 0)
    def _init():
        sup_ref[...] = jnp.zeros(sup_ref.shape, sup_ref.dtype)

    r = rects_ref[...]
    khi_i, klo_i = _cell_keys(r[:, 0:1], r[:, 1:2], r[:, 2:3], r[:, 3:4])
    rt = rects_t_ref[...]
    khi_j, klo_j = _cell_keys(rt[0:1, :], rt[1:2, :], rt[2:3, :], rt[3:4, :])
    ci = conf_ref[...]
    cj = conf_t_ref[...]
    dom = (khi_i == khi_j) & (klo_i == klo_j) & (cj > ci)
    any_dom = jnp.max(dom.astype(jnp.float32), axis=1, keepdims=True)
    sup_ref[...] = jnp.maximum(sup_ref[...], any_dom)


def kernel(rects, conf):
    n = rects.shape[0]
    rects = rects.astype(jnp.float32)
    conf = conf.astype(jnp.float32)

    nr = ((n + _TI - 1) // _TI) * _TI
    nc = ((n + _TJ - 1) // _TJ) * _TJ

    rects_r = jnp.pad(rects, ((0, nr - n), (0, 0)))
    conf_r = jnp.pad(conf, (0, nr - n)).reshape(nr, 1)
    rects_c = jnp.pad(rects, ((0, nc - n), (0, 0)))
    conf_c = jnp.pad(conf, (0, nc - n), constant_values=-1.0).reshape(1, nc)
    rects_t = rects_c.T

    grid = (nr // _TI, nc // _TJ)
    sup = pl.pallas_call(
        _suppress_kernel,
        grid=grid,
        in_specs=[
            pl.BlockSpec((_TI, 4), lambda r, c: (r, r * 0)),
            pl.BlockSpec((4, _TJ), lambda r, c: (c * 0, c)),
            pl.BlockSpec((_TI, 1), lambda r, c: (r, r * 0)),
            pl.BlockSpec((1, _TJ), lambda r, c: (c * 0, c)),
        ],
        out_specs=pl.BlockSpec((_TI, 1), lambda r, c: (r, r * 0)),
        out_shape=jax.ShapeDtypeStruct((nr, 1), jnp.float32),
    )(rects_r, rects_t, conf_r, conf_c)

    keep = 1.0 - sup[:n, 0]
    out = jnp.concatenate([rects * keep[:, None], (conf * keep)[:, None]], axis=1)
    return out
